# baseline (speedup 1.0000x reference)
import functools

import jax
import jax.numpy as jnp
from jax import lax
from jax.experimental import pallas as pl
from jax.experimental.pallas import tpu as pltpu
from jax.experimental.pallas import tpu_sc as plsc

_N = 10000
_E = 320000
_F = 128
_NHID = 256
_NCLASS = 64
_EPS = 0.03

_NC = 2
_NS = 16
_NW = _NC * _NS
_CHUNK = 80
_NCH = 128
_E_PER_W = _NCH * _CHUNK
_E_PAD = _E_PER_W * _NW
_SRC_PER_W = _E_PER_W + 2 * _CHUNK
_N_PAD = 10240
_ROWS_PER_TILE = _N_PAD // _NS


def _sc_agg_body(feat_hbm, src_hbm, dst_hbm, out_hbm,
                 src_v, dst_v, rows0_v, rows1_v, agg_sh, sem0, sem1, sem_i):
    c = lax.axis_index("c")
    s = lax.axis_index("s")
    wid = c * _NS + s

    icp0 = pltpu.async_copy(src_hbm.at[wid], src_v, sem_i)
    icp1 = pltpu.async_copy(dst_hbm.at[wid], dst_v, sem_i)

    zv = jnp.zeros((16,), jnp.float32)

    def zrow(i, carry):
        for k in range(_F // 16):
            rows0_v[i, pl.ds(k * 16, 16)] = zv
        return carry

    lax.fori_loop(0, _CHUNK, zrow, 0)
    base_rows = s * _ROWS_PER_TILE
    for k in range(_ROWS_PER_TILE // _CHUNK):
        pltpu.sync_copy(rows0_v, agg_sh.at[pl.ds(base_rows + k * _CHUNK, _CHUNK)])
    icp0.wait()
    icp1.wait()
    plsc.subcore_barrier()

    rows = (rows0_v, rows1_v)
    sems = (sem0, sem1)

    def _gather(g, b):
        return pltpu.async_copy(
            feat_hbm.at[src_v.at[pl.ds(g * _CHUNK, _CHUNK)]], rows[b], sems[b])

    _gather(0, 0)
    _gather(1, 1)

    @pl.loop(0, _NCH, step=2)
    def _edge_loop(j):
        for b in range(2):
            g = j + b
            pltpu.make_async_copy(
                feat_hbm.at[src_v.at[pl.ds(g * _CHUNK, _CHUNK)]],
                rows[b], sems[b]).wait()
            pltpu.sync_copy(rows[b], agg_sh.at[dst_v.at[g]], add=True)
            _gather(g + 2, b)

    for b in range(2):
        pltpu.make_async_copy(
            feat_hbm.at[src_v.at[pl.ds(_NCH * _CHUNK, _CHUNK)]],
            rows[b], sems[b]).wait()

    plsc.subcore_barrier()

    sl = pl.ds(base_rows, _ROWS_PER_TILE)
    pltpu.sync_copy(agg_sh.at[sl], out_hbm.at[c, sl])


_sc_agg = functools.partial(
    pl.kernel,
    out_type=jax.ShapeDtypeStruct((_NC, _N_PAD, _F), jnp.float32),
    mesh=plsc.VectorSubcoreMesh(core_axis_name="c", subcore_axis_name="s"),
    scratch_types=[
        pltpu.VMEM((_SRC_PER_W,), jnp.int32),
        pltpu.VMEM((_NCH, _CHUNK), jnp.int32),
        pltpu.VMEM((_CHUNK, _F), jnp.float32),
        pltpu.VMEM((_CHUNK, _F), jnp.float32),
        pltpu.VMEM_SHARED((_N_PAD, _F), jnp.float32),
        pltpu.SemaphoreType.DMA,
        pltpu.SemaphoreType.DMA,
        pltpu.SemaphoreType.DMA,
    ],
)(_sc_agg_body)


_MB = 1000


def _mlp_body(feat_ref, agg_ref, w2_ref, b2_ref, w3_ref, b3_ref, out_ref):
    h = (1.0 + _EPS) * feat_ref[...] + agg_ref[0] + agg_ref[1]
    x = jnp.dot(h, w2_ref[...], preferred_element_type=jnp.float32,
                precision=lax.Precision.HIGHEST)
    x = jnp.maximum(x + b2_ref[...], 0.0)
    y = jnp.dot(x, w3_ref[...], preferred_element_type=jnp.float32,
                precision=lax.Precision.HIGHEST)
    y = y + b3_ref[...]
    m = jnp.max(y, axis=1, keepdims=True)
    e = jnp.exp(y - m)
    out_ref[...] = (y - m) - jnp.log(jnp.sum(e, axis=1, keepdims=True))


def kernel(feat, edge_index, W2, b2, W3, b3):
    pad = _E_PAD - _E
    pad_src = jnp.zeros((pad,), jnp.int32)
    pad_dst = _N + (jnp.arange(pad, dtype=jnp.int32) % (_N_PAD - _N))
    src = jnp.concatenate([edge_index[0], pad_src]).reshape(_NW, _E_PER_W)
    src = jnp.pad(src, ((0, 0), (0, _SRC_PER_W - _E_PER_W)))
    dst = jnp.concatenate([edge_index[1], pad_dst]).reshape(_NW, _NCH, _CHUNK)
    agg = _sc_agg(feat, src, dst)

    grid = (_N // _MB,)
    out = pl.pallas_call(
        _mlp_body,
        grid=grid,
        in_specs=[
            pl.BlockSpec((_MB, _F), lambda i: (i, 0)),
            pl.BlockSpec((_NC, _MB, _F), lambda i: (0, i, 0)),
            pl.BlockSpec((_F, _NHID), lambda i: (0, 0)),
            pl.BlockSpec((1, _NHID), lambda i: (0, 0)),
            pl.BlockSpec((_NHID, _NCLASS), lambda i: (0, 0)),
            pl.BlockSpec((1, _NCLASS), lambda i: (0, 0)),
        ],
        out_specs=pl.BlockSpec((_MB, _NCLASS), lambda i: (i, 0)),
        out_shape=jax.ShapeDtypeStruct((_N, _NCLASS), jnp.float32),
    )(feat, agg, W2, b2.reshape(1, _NHID), W3, b3.reshape(1, _NCLASS))
    return out


# speedup vs baseline: 2.8601x; 2.8601x over previous
import functools

import jax
import jax.numpy as jnp
from jax import lax
from jax.experimental import pallas as pl
from jax.experimental.pallas import tpu as pltpu
from jax.experimental.pallas import tpu_sc as plsc

_N = 10000
_E = 320000
_F = 128
_NHID = 256
_NCLASS = 64
_EPS = 0.03

_NC = 2
_NS = 16
_NW = _NC * _NS
_CHUNK = 80
_NCH = 125
_E_PER_W = _NCH * _CHUNK
_SRC_PER_W = _E_PER_W
_N_PAD = 10240
_ROWS_PER_TILE = _N_PAD // _NS


def _sc_agg_body(feat_hbm, src_hbm, dst_hbm, out_hbm,
                 src_v, dst_v, rows0_v, rows1_v, agg_sh, sem0, sem1, sem_i):
    c = lax.axis_index("c")
    s = lax.axis_index("s")
    wid = c * _NS + s

    icp0 = pltpu.async_copy(src_hbm.at[wid], src_v, sem_i)
    icp1 = pltpu.async_copy(dst_hbm.at[wid], dst_v, sem_i)

    zv = jnp.zeros((16,), jnp.float32)

    def zrow(i, carry):
        for k in range(_F // 16):
            rows0_v[i, pl.ds(k * 16, 16)] = zv
        return carry

    lax.fori_loop(0, _CHUNK, zrow, 0)
    base_rows = s * _ROWS_PER_TILE
    for k in range(_ROWS_PER_TILE // _CHUNK):
        pltpu.sync_copy(rows0_v, agg_sh.at[pl.ds(base_rows + k * _CHUNK, _CHUNK)])
    icp0.wait()
    icp1.wait()
    plsc.subcore_barrier()

    def body(j, carry):
        pltpu.async_copy(
            feat_hbm.at[src_v.at[pl.ds(j * _CHUNK, _CHUNK)]],
            rows0_v, sem0).wait()
        pltpu.sync_copy(rows0_v, agg_sh.at[dst_v.at[j]], add=True)
        return carry

    lax.fori_loop(0, _NCH, body, 0)
    plsc.subcore_barrier()

    sl = pl.ds(base_rows, _ROWS_PER_TILE)
    pltpu.sync_copy(agg_sh.at[sl], out_hbm.at[c, sl])


_sc_agg = functools.partial(
    pl.kernel,
    out_type=jax.ShapeDtypeStruct((_NC, _N_PAD, _F), jnp.float32),
    mesh=plsc.VectorSubcoreMesh(core_axis_name="c", subcore_axis_name="s"),
    scratch_types=[
        pltpu.VMEM((_SRC_PER_W,), jnp.int32),
        pltpu.VMEM((_NCH, _CHUNK), jnp.int32),
        pltpu.VMEM((_CHUNK, _F), jnp.float32),
        pltpu.VMEM((_CHUNK, _F), jnp.float32),
        pltpu.VMEM_SHARED((_N_PAD, _F), jnp.float32),
        pltpu.SemaphoreType.DMA,
        pltpu.SemaphoreType.DMA,
        pltpu.SemaphoreType.DMA,
    ],
)(_sc_agg_body)


_MB = 1000


def _mlp_body(feat_ref, agg_ref, w2_ref, b2_ref, w3_ref, b3_ref, out_ref):
    h = (1.0 + _EPS) * feat_ref[...] + agg_ref[0] + agg_ref[1]
    x = jnp.dot(h, w2_ref[...], preferred_element_type=jnp.float32,
                precision=lax.Precision.HIGHEST)
    x = jnp.maximum(x + b2_ref[...], 0.0)
    y = jnp.dot(x, w3_ref[...], preferred_element_type=jnp.float32,
                precision=lax.Precision.HIGHEST)
    y = y + b3_ref[...]
    m = jnp.max(y, axis=1, keepdims=True)
    e = jnp.exp(y - m)
    out_ref[...] = (y - m) - jnp.log(jnp.sum(e, axis=1, keepdims=True))


def kernel(feat, edge_index, W2, b2, W3, b3):
    src = edge_index[0].reshape(_NW, _SRC_PER_W)
    dst = edge_index[1].reshape(_NW, _NCH, _CHUNK)
    agg = _sc_agg(feat, src, dst)

    grid = (_N // _MB,)
    out = pl.pallas_call(
        _mlp_body,
        grid=grid,
        in_specs=[
            pl.BlockSpec((_MB, _F), lambda i: (i, 0)),
            pl.BlockSpec((_NC, _MB, _F), lambda i: (0, i, 0)),
            pl.BlockSpec((_F, _NHID), lambda i: (0, 0)),
            pl.BlockSpec((1, _NHID), lambda i: (0, 0)),
            pl.BlockSpec((_NHID, _NCLASS), lambda i: (0, 0)),
            pl.BlockSpec((1, _NCLASS), lambda i: (0, 0)),
        ],
        out_specs=pl.BlockSpec((_MB, _NCLASS), lambda i: (i, 0)),
        out_shape=jax.ShapeDtypeStruct((_N, _NCLASS), jnp.float32),
    )(feat, agg, W2, b2.reshape(1, _NHID), W3, b3.reshape(1, _NCLASS))
    return out


# speedup vs baseline: 3.4305x; 1.1994x over previous
import functools

import jax
import jax.numpy as jnp
from jax import lax
from jax.experimental import pallas as pl
from jax.experimental.pallas import tpu as pltpu
from jax.experimental.pallas import tpu_sc as plsc

_N = 10000
_E = 320000
_F = 128
_NHID = 256
_NCLASS = 64
_EPS = 0.03

_NC = 2
_NS = 16
_NW = _NC * _NS
_CHUNK = 80
_NCH = 125
_E_PER_W = _NCH * _CHUNK
_SRC_PER_W = _E_PER_W
_N_PAD = 10240
_ROWS_PER_TILE = _N_PAD // _NS


def _sc_agg_body(feat_hbm, src_hbm, dst_hbm, out_hbm,
                 src_v, dst_v, rows0_v, rows1_v, agg_sh, sem0, sem1, sem_i):
    c = lax.axis_index("c")
    s = lax.axis_index("s")
    wid = c * _NS + s

    icp0 = pltpu.async_copy(src_hbm.at[wid], src_v, sem_i)
    icp1 = pltpu.async_copy(dst_hbm.at[wid], dst_v, sem_i)

    zv = jnp.zeros((16,), jnp.float32)

    def zrow(i, carry):
        for k in range(_F // 16):
            rows0_v[i, pl.ds(k * 16, 16)] = zv
        return carry

    lax.fori_loop(0, _CHUNK, zrow, 0)
    base_rows = s * _ROWS_PER_TILE
    for k in range(_ROWS_PER_TILE // _CHUNK):
        pltpu.sync_copy(rows0_v, agg_sh.at[pl.ds(base_rows + k * _CHUNK, _CHUNK)])
    icp0.wait()
    icp1.wait()
    plsc.subcore_barrier()

    @pl.loop(0, _NCH - 1, step=2)
    def _pair(j):
        cp0 = pltpu.async_copy(
            feat_hbm.at[src_v.at[pl.ds(j * _CHUNK, _CHUNK)]], rows0_v, sem0)
        cp1 = pltpu.async_copy(
            feat_hbm.at[src_v.at[pl.ds((j + 1) * _CHUNK, _CHUNK)]], rows1_v, sem1)
        cp0.wait()
        pltpu.sync_copy(rows0_v, agg_sh.at[dst_v.at[j]], add=True)
        cp1.wait()
        pltpu.sync_copy(rows1_v, agg_sh.at[dst_v.at[j + 1]], add=True)

    pltpu.async_copy(
        feat_hbm.at[src_v.at[pl.ds((_NCH - 1) * _CHUNK, _CHUNK)]],
        rows0_v, sem0).wait()
    pltpu.sync_copy(rows0_v, agg_sh.at[dst_v.at[_NCH - 1]], add=True)
    plsc.subcore_barrier()

    sl = pl.ds(base_rows, _ROWS_PER_TILE)
    pltpu.sync_copy(agg_sh.at[sl], out_hbm.at[c, sl])


_sc_agg = functools.partial(
    pl.kernel,
    out_type=jax.ShapeDtypeStruct((_NC, _N_PAD, _F), jnp.float32),
    mesh=plsc.VectorSubcoreMesh(core_axis_name="c", subcore_axis_name="s"),
    scratch_types=[
        pltpu.VMEM((_SRC_PER_W,), jnp.int32),
        pltpu.VMEM((_NCH, _CHUNK), jnp.int32),
        pltpu.VMEM((_CHUNK, _F), jnp.float32),
        pltpu.VMEM((_CHUNK, _F), jnp.float32),
        pltpu.VMEM_SHARED((_N_PAD, _F), jnp.float32),
        pltpu.SemaphoreType.DMA,
        pltpu.SemaphoreType.DMA,
        pltpu.SemaphoreType.DMA,
    ],
)(_sc_agg_body)


_MB = 1000


def _mlp_body(feat_ref, agg_ref, w2_ref, b2_ref, w3_ref, b3_ref, out_ref):
    h = (1.0 + _EPS) * feat_ref[...] + agg_ref[0] + agg_ref[1]
    x = jnp.dot(h, w2_ref[...], preferred_element_type=jnp.float32,
                precision=lax.Precision.HIGHEST)
    x = jnp.maximum(x + b2_ref[...], 0.0)
    y = jnp.dot(x, w3_ref[...], preferred_element_type=jnp.float32,
                precision=lax.Precision.HIGHEST)
    y = y + b3_ref[...]
    m = jnp.max(y, axis=1, keepdims=True)
    e = jnp.exp(y - m)
    out_ref[...] = (y - m) - jnp.log(jnp.sum(e, axis=1, keepdims=True))


def kernel(feat, edge_index, W2, b2, W3, b3):
    src = edge_index[0].reshape(_NW, _SRC_PER_W)
    dst = edge_index[1].reshape(_NW, _NCH, _CHUNK)
    agg = _sc_agg(feat, src, dst)

    grid = (_N // _MB,)
    out = pl.pallas_call(
        _mlp_body,
        grid=grid,
        in_specs=[
            pl.BlockSpec((_MB, _F), lambda i: (i, 0)),
            pl.BlockSpec((_NC, _MB, _F), lambda i: (0, i, 0)),
            pl.BlockSpec((_F, _NHID), lambda i: (0, 0)),
            pl.BlockSpec((1, _NHID), lambda i: (0, 0)),
            pl.BlockSpec((_NHID, _NCLASS), lambda i: (0, 0)),
            pl.BlockSpec((1, _NCLASS), lambda i: (0, 0)),
        ],
        out_specs=pl.BlockSpec((_MB, _NCLASS), lambda i: (i, 0)),
        out_shape=jax.ShapeDtypeStruct((_N, _NCLASS), jnp.float32),
    )(feat, agg, W2, b2.reshape(1, _NHID), W3, b3.reshape(1, _NCLASS))
    return out


# speedup vs baseline: 3.4880x; 1.0168x over previous
import functools

import jax
import jax.numpy as jnp
from jax import lax
from jax.experimental import pallas as pl
from jax.experimental.pallas import tpu as pltpu
from jax.experimental.pallas import tpu_sc as plsc

_N = 10000
_E = 320000
_F = 128
_NHID = 256
_NCLASS = 64
_EPS = 0.03

_NC = 2
_NS = 16
_NW = _NC * _NS
_CHUNK = 80
_NCH = 125
_E_PER_W = _NCH * _CHUNK
_SRC_PER_W = _E_PER_W
_N_PAD = 10240
_ROWS_PER_TILE = _N_PAD // _NS


def _sc_agg_body(feat_hbm, src_hbm, dst_hbm, out_hbm,
                 src_v, dst_v, rows0_v, rows1_v, agg_sh,
                 sem0, sem1, sem_i, sem_s0, sem_s1):
    c = lax.axis_index("c")
    s = lax.axis_index("s")
    wid = c * _NS + s

    icp0 = pltpu.async_copy(src_hbm.at[wid], src_v, sem_i)
    icp1 = pltpu.async_copy(dst_hbm.at[wid], dst_v, sem_i)

    zv = jnp.zeros((16,), jnp.float32)

    def zrow(i, carry):
        for k in range(_F // 16):
            rows0_v[i, pl.ds(k * 16, 16)] = zv
        return carry

    lax.fori_loop(0, _CHUNK, zrow, 0)
    base_rows = s * _ROWS_PER_TILE
    for k in range(_ROWS_PER_TILE // _CHUNK):
        pltpu.sync_copy(rows0_v, agg_sh.at[pl.ds(base_rows + k * _CHUNK, _CHUNK)])
    icp0.wait()
    icp1.wait()
    plsc.subcore_barrier()

    @pl.loop(0, _NCH - 1, step=2)
    def _pair(j):
        cp0 = pltpu.async_copy(
            feat_hbm.at[src_v.at[pl.ds(j * _CHUNK, _CHUNK)]], rows0_v, sem0)
        cp1 = pltpu.async_copy(
            feat_hbm.at[src_v.at[pl.ds((j + 1) * _CHUNK, _CHUNK)]], rows1_v, sem1)
        cp0.wait()
        sc0 = pltpu.async_copy(rows0_v, agg_sh.at[dst_v.at[j]], sem_s0, add=True)
        cp1.wait()
        sc1 = pltpu.async_copy(rows1_v, agg_sh.at[dst_v.at[j + 1]], sem_s1, add=True)
        sc0.wait()
        sc1.wait()

    pltpu.async_copy(
        feat_hbm.at[src_v.at[pl.ds((_NCH - 1) * _CHUNK, _CHUNK)]],
        rows0_v, sem0).wait()
    pltpu.sync_copy(rows0_v, agg_sh.at[dst_v.at[_NCH - 1]], add=True)
    plsc.subcore_barrier()

    sl = pl.ds(base_rows, _ROWS_PER_TILE)
    pltpu.sync_copy(agg_sh.at[sl], out_hbm.at[c, sl])


_sc_agg = functools.partial(
    pl.kernel,
    out_type=jax.ShapeDtypeStruct((_NC, _N_PAD, _F), jnp.float32),
    mesh=plsc.VectorSubcoreMesh(core_axis_name="c", subcore_axis_name="s"),
    scratch_types=[
        pltpu.VMEM((_SRC_PER_W,), jnp.int32),
        pltpu.VMEM((_NCH, _CHUNK), jnp.int32),
        pltpu.VMEM((_CHUNK, _F), jnp.float32),
        pltpu.VMEM((_CHUNK, _F), jnp.float32),
        pltpu.VMEM_SHARED((_N_PAD, _F), jnp.float32),
        pltpu.SemaphoreType.DMA,
        pltpu.SemaphoreType.DMA,
        pltpu.SemaphoreType.DMA,
        pltpu.SemaphoreType.DMA,
        pltpu.SemaphoreType.DMA,
    ],
)(_sc_agg_body)


_MB = 1000


def _mlp_body(feat_ref, agg_ref, w2_ref, b2_ref, w3_ref, b3_ref, out_ref):
    h = (1.0 + _EPS) * feat_ref[...] + agg_ref[0] + agg_ref[1]
    x = jnp.dot(h, w2_ref[...], preferred_element_type=jnp.float32,
                precision=lax.Precision.HIGHEST)
    x = jnp.maximum(x + b2_ref[...], 0.0)
    y = jnp.dot(x, w3_ref[...], preferred_element_type=jnp.float32,
                precision=lax.Precision.HIGHEST)
    y = y + b3_ref[...]
    m = jnp.max(y, axis=1, keepdims=True)
    e = jnp.exp(y - m)
    out_ref[...] = (y - m) - jnp.log(jnp.sum(e, axis=1, keepdims=True))


def kernel(feat, edge_index, W2, b2, W3, b3):
    src = edge_index[0].reshape(_NW, _SRC_PER_W)
    dst = edge_index[1].reshape(_NW, _NCH, _CHUNK)
    agg = _sc_agg(feat, src, dst)

    grid = (_N // _MB,)
    out = pl.pallas_call(
        _mlp_body,
        grid=grid,
        in_specs=[
            pl.BlockSpec((_MB, _F), lambda i: (i, 0)),
            pl.BlockSpec((_NC, _MB, _F), lambda i: (0, i, 0)),
            pl.BlockSpec((_F, _NHID), lambda i: (0, 0)),
            pl.BlockSpec((1, _NHID), lambda i: (0, 0)),
            pl.BlockSpec((_NHID, _NCLASS), lambda i: (0, 0)),
            pl.BlockSpec((1, _NCLASS), lambda i: (0, 0)),
        ],
        out_specs=pl.BlockSpec((_MB, _NCLASS), lambda i: (i, 0)),
        out_shape=jax.ShapeDtypeStruct((_N, _NCLASS), jnp.float32),
    )(feat, agg, W2, b2.reshape(1, _NHID), W3, b3.reshape(1, _NCLASS))
    return out


# speedup vs baseline: 3.9229x; 1.1247x over previous
import functools

import jax
import jax.numpy as jnp
from jax import lax
from jax.experimental import pallas as pl
from jax.experimental.pallas import tpu as pltpu
from jax.experimental.pallas import tpu_sc as plsc

_N = 10000
_E = 320000
_F = 128
_NHID = 256
_NCLASS = 64
_EPS = 0.03

_NC = 2
_NS = 16
_NW = _NC * _NS
_CHUNK = 80
_NCH = 125
_E_PER_W = _NCH * _CHUNK
_SRC_PER_W = _E_PER_W
_N_PAD = 10240
_ROWS_PER_TILE = _N_PAD // _NS


def _sc_agg_body(feat_hbm, src_hbm, dst_hbm, out_hbm,
                 src_v, dst_v, rows0_v, rows1_v, agg_sh,
                 sem0, sem1, sem_i, sem_s0, sem_s1):
    c = lax.axis_index("c")
    s = lax.axis_index("s")
    wid = c * _NS + s

    icp0 = pltpu.async_copy(src_hbm.at[wid], src_v, sem_i)
    icp1 = pltpu.async_copy(dst_hbm.at[wid], dst_v, sem_i)

    zv = jnp.zeros((16,), jnp.float32)

    def zrow(i, carry):
        for k in range(_F // 16):
            rows0_v[i, pl.ds(k * 16, 16)] = zv
        return carry

    lax.fori_loop(0, _CHUNK, zrow, 0)
    base_rows = s * _ROWS_PER_TILE
    for k in range(_ROWS_PER_TILE // _CHUNK):
        pltpu.sync_copy(rows0_v, agg_sh.at[pl.ds(base_rows + k * _CHUNK, _CHUNK)])
    icp0.wait()
    icp1.wait()
    plsc.subcore_barrier()

    @pl.loop(0, _NCH - 1, step=2)
    def _pair(j):
        cp0 = pltpu.async_copy(
            feat_hbm.at[src_v.at[pl.ds(j * _CHUNK, _CHUNK)]], rows0_v, sem0)
        cp1 = pltpu.async_copy(
            feat_hbm.at[src_v.at[pl.ds((j + 1) * _CHUNK, _CHUNK)]], rows1_v, sem1)
        cp0.wait()
        sc0 = pltpu.async_copy(rows0_v, agg_sh.at[dst_v.at[j]], sem_s0, add=True)
        cp1.wait()
        sc1 = pltpu.async_copy(rows1_v, agg_sh.at[dst_v.at[j + 1]], sem_s1, add=True)
        sc0.wait()
        sc1.wait()

    pltpu.async_copy(
        feat_hbm.at[src_v.at[pl.ds((_NCH - 1) * _CHUNK, _CHUNK)]],
        rows0_v, sem0).wait()
    pltpu.sync_copy(rows0_v, agg_sh.at[dst_v.at[_NCH - 1]], add=True)
    plsc.subcore_barrier()

    sl = pl.ds(base_rows, _ROWS_PER_TILE)
    pltpu.sync_copy(agg_sh.at[sl], out_hbm.at[c, sl])


_sc_agg = functools.partial(
    pl.kernel,
    out_type=jax.ShapeDtypeStruct((_NC, _N_PAD, _F), jnp.float32),
    mesh=plsc.VectorSubcoreMesh(core_axis_name="c", subcore_axis_name="s"),
    scratch_types=[
        pltpu.VMEM((_SRC_PER_W,), jnp.int32),
        pltpu.VMEM((_NCH, _CHUNK), jnp.int32),
        pltpu.VMEM((_CHUNK, _F), jnp.float32),
        pltpu.VMEM((_CHUNK, _F), jnp.float32),
        pltpu.VMEM_SHARED((_N_PAD, _F), jnp.float32),
        pltpu.SemaphoreType.DMA,
        pltpu.SemaphoreType.DMA,
        pltpu.SemaphoreType.DMA,
        pltpu.SemaphoreType.DMA,
        pltpu.SemaphoreType.DMA,
    ],
)(_sc_agg_body)


_MB = 2000


def _mlp_body(feat_ref, agg_ref, w2_ref, b2_ref, w3_ref, b3_ref, out_ref):
    h = (1.0 + _EPS) * feat_ref[...] + agg_ref[0] + agg_ref[1]
    x = jnp.dot(h, w2_ref[...], preferred_element_type=jnp.float32)
    x = jnp.maximum(x + b2_ref[...], 0.0)
    y = jnp.dot(x, w3_ref[...], preferred_element_type=jnp.float32)
    y = y + b3_ref[...]
    m = jnp.max(y, axis=1, keepdims=True)
    e = jnp.exp(y - m)
    out_ref[...] = (y - m) - jnp.log(jnp.sum(e, axis=1, keepdims=True))


def kernel(feat, edge_index, W2, b2, W3, b3):
    src = edge_index[0].reshape(_NW, _SRC_PER_W)
    dst = edge_index[1].reshape(_NW, _NCH, _CHUNK)
    agg = _sc_agg(feat, src, dst)

    grid = (_N // _MB,)
    out = pl.pallas_call(
        _mlp_body,
        grid=grid,
        in_specs=[
            pl.BlockSpec((_MB, _F), lambda i: (i, 0)),
            pl.BlockSpec((_NC, _MB, _F), lambda i: (0, i, 0)),
            pl.BlockSpec((_F, _NHID), lambda i: (0, 0)),
            pl.BlockSpec((1, _NHID), lambda i: (0, 0)),
            pl.BlockSpec((_NHID, _NCLASS), lambda i: (0, 0)),
            pl.BlockSpec((1, _NCLASS), lambda i: (0, 0)),
        ],
        out_specs=pl.BlockSpec((_MB, _NCLASS), lambda i: (i, 0)),
        out_shape=jax.ShapeDtypeStruct((_N, _NCLASS), jnp.float32),
    )(feat, agg, W2, b2.reshape(1, _NHID), W3, b3.reshape(1, _NCLASS))
    return out
